# Initial kernel scaffold; baseline (speedup 1.0000x reference)
import jax
import jax.numpy as jnp
from jax.experimental import pallas as pl


def kernel(hidden_states, gate_weight, gate_up_weights, down_weights):
    raise NotImplementedError("write your pallas kernel here")


# speedup vs baseline: 1.4429x; 1.4429x over previous
import functools

import jax
import jax.numpy as jnp
from jax.experimental import pallas as pl
from jax.experimental.pallas import tpu as pltpu

NUM_EXPERTS = 8
TOP_K = 2
HIDDEN = 1024
FFN = 2048
T = 2048
NF = 4
FC = FFN // NF


def _moe_body(x_ref, gwp_ref, gw_ref, uw_ref, dw_ref,
              out_ref, logits_ref, w_scr):
    e = pl.program_id(0)
    f = pl.program_id(1)
    first = jnp.logical_and(e == 0, f == 0)

    lane = jax.lax.broadcasted_iota(jnp.int32, (T, 128), 1)

    @pl.when(first)
    def _router():
        x = x_ref[...]
        logits_full = jax.lax.dot_general(
            x, gwp_ref[...], (((1,), (1,)), ((), ())),
            preferred_element_type=jnp.float32)
        logits_ref[...] = logits_full
        neg = jnp.float32(-1e30)
        lp = jnp.where(lane < NUM_EXPERTS, logits_full, neg)
        m1 = jnp.max(lp, axis=1, keepdims=True)
        idx1 = jnp.min(jnp.where(lp == m1, lane, 12345), axis=1, keepdims=True)
        mask1 = lane == idx1
        lp2 = jnp.where(mask1, neg, lp)
        m2 = jnp.max(lp2, axis=1, keepdims=True)
        idx2 = jnp.min(jnp.where(lp2 == m2, lane, 12345), axis=1, keepdims=True)
        mask2 = lane == idx2
        w1 = 1.0 / (1.0 + jnp.exp(m2 - m1))
        w2 = 1.0 - w1
        w_scr[...] = jnp.where(mask1, w1, 0.0) + jnp.where(mask2, w2, 0.0)

    x = x_ref[...]
    g = jax.lax.dot_general(x, gw_ref[0], (((1,), (1,)), ((), ())),
                            preferred_element_type=jnp.float32)
    u = jax.lax.dot_general(x, uw_ref[0], (((1,), (1,)), ((), ())),
                            preferred_element_type=jnp.float32)
    h = g * (1.0 / (1.0 + jnp.exp(-g))) * u
    o = jax.lax.dot_general(h, dw_ref[0], (((1,), (1,)), ((), ())),
                            preferred_element_type=jnp.float32)
    wcol = jnp.sum(jnp.where(lane == e, w_scr[...], 0.0), axis=1,
                   keepdims=True)
    contrib = wcol * o

    @pl.when(first)
    def _init():
        out_ref[...] = contrib

    @pl.when(jnp.logical_not(first))
    def _acc():
        out_ref[...] += contrib


@functools.partial(jax.jit, static_argnames=())
def kernel(hidden_states, gate_weight, gate_up_weights, down_weights):
    b, s, hd = hidden_states.shape
    x = hidden_states.reshape(-1, hd)
    gwp = jnp.zeros((128, HIDDEN), jnp.float32).at[:NUM_EXPERTS].set(gate_weight)

    grid = (NUM_EXPERTS, NF)
    out, logits_full = pl.pallas_call(
        _moe_body,
        grid=grid,
        in_specs=[
            pl.BlockSpec((T, HIDDEN), lambda e, f: (0, 0)),
            pl.BlockSpec((128, HIDDEN), lambda e, f: (0, 0)),
            pl.BlockSpec((1, FC, HIDDEN), lambda e, f: (e, f, 0)),
            pl.BlockSpec((1, FC, HIDDEN), lambda e, f: (e, f + NF, 0)),
            pl.BlockSpec((1, HIDDEN, FC), lambda e, f: (e, 0, f)),
        ],
        out_specs=[
            pl.BlockSpec((T, HIDDEN), lambda e, f: (0, 0)),
            pl.BlockSpec((T, 128), lambda e, f: (0, 0)),
        ],
        out_shape=[
            jax.ShapeDtypeStruct((T, HIDDEN), jnp.float32),
            jax.ShapeDtypeStruct((T, 128), jnp.float32),
        ],
        scratch_shapes=[pltpu.VMEM((T, 128), jnp.float32)],
    )(x, gwp, gate_up_weights, gate_up_weights, down_weights)

    router_logits = logits_full[:, :NUM_EXPERTS]
    return out.reshape(b, s, hd), router_logits
